# Initial kernel scaffold; baseline (speedup 1.0000x reference)
import jax
import jax.numpy as jnp
from jax.experimental import pallas as pl


def kernel(input_ids, attention_mask, table):
    raise NotImplementedError("write your pallas kernel here")


# speedup vs baseline: 1.6259x; 1.6259x over previous
import functools

import jax
import jax.numpy as jnp
from jax import lax
from jax.experimental import pallas as pl
from jax.experimental.pallas import tpu as pltpu
from jax.experimental.pallas import tpu_sc as plsc


@functools.cache
def _build_gather(n_rows: int, dim: int):
    info = plsc.get_sparse_core_info()
    nc, ns = info.num_cores, info.num_subcores
    nw = nc * ns
    rows_per_w = n_rows // nw
    assert n_rows % (8 * nw) == 0
    ch = 16
    n_chunks = rows_per_w // ch
    mesh = plsc.VectorSubcoreMesh(core_axis_name="c", subcore_axis_name="s")

    def body(ids_hbm, table_hbm, out_hbm, idx_v, rows_v, sem):
        wid = lax.axis_index("s") * nc + lax.axis_index("c")
        base = wid * rows_per_w
        pltpu.sync_copy(ids_hbm.at[pl.ds(base, rows_per_w)], idx_v)

        @pl.loop(0, n_chunks)
        def _(c):
            off = pl.multiple_of(c * ch, ch)
            pltpu.async_copy(
                table_hbm.at[idx_v.at[pl.ds(off, ch)]], rows_v, sem
            ).wait()
            pltpu.sync_copy(rows_v, out_hbm.at[pl.ds(base + off, ch)])

    return pl.kernel(
        body,
        out_type=jax.ShapeDtypeStruct((n_rows, dim), jnp.float32),
        mesh=mesh,
        scratch_types=[
            pltpu.VMEM((rows_per_w,), jnp.int32),
            pltpu.VMEM((ch, dim), jnp.float32),
            pltpu.SemaphoreType.DMA,
        ],
    )


def kernel(input_ids, attention_mask, table):
    b, s = input_ids.shape
    dim = table.shape[1]
    ids = input_ids.reshape(-1).astype(jnp.int32)
    out = _build_gather(b * s, dim)(ids, table)
    return out.reshape(b, s, dim), attention_mask


# speedup vs baseline: 1.7849x; 1.0978x over previous
import functools

import jax
import jax.numpy as jnp
from jax import lax
from jax.experimental import pallas as pl
from jax.experimental.pallas import tpu as pltpu
from jax.experimental.pallas import tpu_sc as plsc


@functools.cache
def _build_gather(n_rows: int, dim: int):
    info = plsc.get_sparse_core_info()
    nc, ns = info.num_cores, info.num_subcores
    nw = nc * ns
    rows_per_w = n_rows // nw
    assert n_rows % (8 * nw) == 0
    ch = 16
    n_chunks = rows_per_w // ch
    assert n_chunks % 2 == 0 and n_chunks >= 4
    mesh = plsc.VectorSubcoreMesh(core_axis_name="c", subcore_axis_name="s")

    def body(ids_hbm, table_hbm, out_hbm, idx_v, rows_v,
             gsem0, gsem1, wsem0, wsem1):
        gsems = (gsem0, gsem1)
        wsems = (wsem0, wsem1)
        wid = lax.axis_index("s") * nc + lax.axis_index("c")
        base = wid * rows_per_w
        pltpu.sync_copy(ids_hbm.at[pl.ds(base, rows_per_w)], idx_v)

        def start_gather(c, b):
            off = pl.multiple_of(c * ch, ch)
            pltpu.async_copy(
                table_hbm.at[idx_v.at[pl.ds(off, ch)]], rows_v.at[b], gsems[b])

        def wait_gather(b):
            pltpu.make_async_copy(
                table_hbm.at[idx_v.at[pl.ds(0, ch)]], rows_v.at[b],
                gsems[b]).wait()

        def start_write(c, b):
            off = pl.multiple_of(c * ch, ch)
            pltpu.async_copy(
                rows_v.at[b], out_hbm.at[pl.ds(base + off, ch)], wsems[b])

        def wait_write(b):
            pltpu.make_async_copy(
                rows_v.at[b], out_hbm.at[pl.ds(base, ch)], wsems[b]).wait()

        start_gather(0, 0)
        start_gather(1, 1)
        wait_gather(0)
        start_write(0, 0)

        @pl.loop(0, (n_chunks - 2) // 2)
        def _(g):
            for k in range(2):
                c = 1 + 2 * g + k
                b = (1 + k) % 2
                wait_write(1 - b)
                start_gather(c + 1, 1 - b)
                wait_gather(b)
                start_write(c, b)

        c_last = n_chunks - 1
        wait_write(0)
        wait_gather(1)
        start_write(c_last, 1)
        wait_write(1)

    return pl.kernel(
        body,
        out_type=jax.ShapeDtypeStruct((n_rows, dim), jnp.float32),
        mesh=mesh,
        scratch_types=[
            pltpu.VMEM((rows_per_w,), jnp.int32),
            pltpu.VMEM((2, ch, dim), jnp.float32),
            pltpu.SemaphoreType.DMA,
            pltpu.SemaphoreType.DMA,
            pltpu.SemaphoreType.DMA,
            pltpu.SemaphoreType.DMA,
        ],
    )


def kernel(input_ids, attention_mask, table):
    b, s = input_ids.shape
    dim = table.shape[1]
    ids = input_ids.reshape(-1).astype(jnp.int32)
    out = _build_gather(b * s, dim)(ids, table)
    return out.reshape(b, s, dim), attention_mask


# speedup vs baseline: 1.7879x; 1.0017x over previous
import functools

import jax
import jax.numpy as jnp
from jax import lax
from jax.experimental import pallas as pl
from jax.experimental.pallas import tpu as pltpu
from jax.experimental.pallas import tpu_sc as plsc

_NBUF = 4
_CH = 8
_LOOK = 2


@functools.cache
def _build_gather(n_rows: int, dim: int):
    info = plsc.get_sparse_core_info()
    nc, ns = info.num_cores, info.num_subcores
    nw = nc * ns
    rows_per_w = n_rows // nw
    assert n_rows % (8 * nw) == 0
    n_chunks = rows_per_w // _CH
    assert (n_chunks - 2 * _LOOK) % _NBUF == 0
    mesh = plsc.VectorSubcoreMesh(core_axis_name="c", subcore_axis_name="s")

    def body(ids_hbm, table_hbm, out_hbm, idx_v, rows_v, *sems):
        gsems = sems[:_NBUF]
        wsems = sems[_NBUF:]
        wid = lax.axis_index("s") * nc + lax.axis_index("c")
        base = wid * rows_per_w
        pltpu.sync_copy(ids_hbm.at[pl.ds(base, rows_per_w)], idx_v)

        def start_gather(c, b):
            off = pl.multiple_of(c * _CH, _CH)
            pltpu.async_copy(
                table_hbm.at[idx_v.at[pl.ds(off, _CH)]], rows_v.at[b],
                gsems[b])

        def wait_gather(b):
            pltpu.make_async_copy(
                table_hbm.at[idx_v.at[pl.ds(0, _CH)]], rows_v.at[b],
                gsems[b]).wait()

        def start_write(c, b):
            off = pl.multiple_of(c * _CH, _CH)
            pltpu.async_copy(
                rows_v.at[b], out_hbm.at[pl.ds(base + off, _CH)], wsems[b])

        def wait_write(b):
            pltpu.make_async_copy(
                rows_v.at[b], out_hbm.at[pl.ds(base, _CH)], wsems[b]).wait()

        for c in range(_LOOK):
            start_gather(c, c % _NBUF)
        for c in range(_LOOK):
            start_gather(c + _LOOK, (c + _LOOK) % _NBUF)
            wait_gather(c % _NBUF)
            start_write(c, c % _NBUF)

        @pl.loop(0, (n_chunks - 2 * _LOOK) // _NBUF)
        def _(g):
            for k in range(_NBUF):
                c = _LOOK + _NBUF * g + k
                b = (_LOOK + k) % _NBUF
                bn = (b + _LOOK) % _NBUF
                wait_write(bn)
                start_gather(c + _LOOK, bn)
                wait_gather(b)
                start_write(c, b)

        for c in range(n_chunks - _LOOK, n_chunks):
            b = c % _NBUF
            wait_gather(b)
            start_write(c, b)
        for c in range(n_chunks - _NBUF, n_chunks):
            wait_write(c % _NBUF)

    return pl.kernel(
        body,
        out_type=jax.ShapeDtypeStruct((n_rows, dim), jnp.float32),
        mesh=mesh,
        scratch_types=[
            pltpu.VMEM((rows_per_w,), jnp.int32),
            pltpu.VMEM((_NBUF, _CH, dim), jnp.float32),
        ] + [pltpu.SemaphoreType.DMA] * (2 * _NBUF),
    )


def kernel(input_ids, attention_mask, table):
    b, s = input_ids.shape
    dim = table.shape[1]
    ids = input_ids.reshape(-1).astype(jnp.int32)
    out = _build_gather(b * s, dim)(ids, table)
    return out.reshape(b, s, dim), attention_mask


# speedup vs baseline: 1.8371x; 1.0275x over previous
import functools

import jax
import jax.numpy as jnp
from jax import lax
from jax.experimental import pallas as pl
from jax.experimental.pallas import tpu as pltpu
from jax.experimental.pallas import tpu_sc as plsc

_CH = 8
_SCH = 8
_SROWS = 256


@functools.cache
def _build_gather(n_rows: int, dim: int):
    info = plsc.get_sparse_core_info()
    nc, ns = info.num_cores, info.num_subcores
    nw = nc * ns
    rows_per_w = n_rows // nw
    assert n_rows % (8 * nw) == 0
    t_rows = rows_per_w - _SROWS
    n_chunks = t_rows // _CH
    s_chunks = _SROWS // _SCH
    n_loop = (n_chunks - 2) // 2
    assert 2 * n_loop == s_chunks - 2
    mesh = plsc.VectorSubcoreMesh(core_axis_name="c", subcore_axis_name="s")

    def body(ids_hbm, table_hbm, out_hbm, idx_v, rows_v, stage_sh, *sems):
        gsems = sems[0:2]
        wsems = sems[2:4]
        ssems = sems[4:6]
        fsems = sems[6:8]
        sid = lax.axis_index("s")
        wid = sid * nc + lax.axis_index("c")
        base = wid * rows_per_w
        pltpu.sync_copy(ids_hbm.at[pl.ds(base, rows_per_w)], idx_v)

        def start_gather(c, b):
            off = pl.multiple_of(_SROWS + c * _CH, _CH)
            pltpu.async_copy(
                table_hbm.at[idx_v.at[pl.ds(off, _CH)]], rows_v.at[b],
                gsems[b])

        def wait_gather(b):
            pltpu.make_async_copy(
                table_hbm.at[idx_v.at[pl.ds(0, _CH)]], rows_v.at[b],
                gsems[b]).wait()

        def start_write(c, b):
            off = pl.multiple_of(_SROWS + c * _CH, _CH)
            pltpu.async_copy(
                rows_v.at[b], out_hbm.at[pl.ds(base + off, _CH)], wsems[b])

        def wait_write(b):
            pltpu.make_async_copy(
                rows_v.at[b], out_hbm.at[pl.ds(base, _CH)], wsems[b]).wait()

        def stream_step(c, b):
            wait_write(1 - b)
            start_gather(c + 1, 1 - b)
            wait_gather(b)
            start_write(c, b)

        def start_stage(j, sl):
            ids8 = idx_v[pl.ds(pl.multiple_of(j * _SCH, _SCH), _SCH)]
            for r in range(_SCH):
                row = ids8[r]
                pltpu.async_copy(
                    table_hbm.at[pl.ds(row, 1)],
                    stage_sh.at[sid, sl, pl.ds(r, 1)], ssems[sl])

        def wait_stage(sl):
            pltpu.make_async_copy(
                table_hbm.at[pl.ds(0, _SCH)], stage_sh.at[sid, sl],
                ssems[sl]).wait()

        def start_flush(j, sl):
            off = pl.multiple_of(j * _SCH, _SCH)
            pltpu.async_copy(
                stage_sh.at[sid, sl], out_hbm.at[pl.ds(base + off, _SCH)],
                fsems[sl])

        def wait_flush(sl):
            pltpu.make_async_copy(
                stage_sh.at[sid, 0], out_hbm.at[pl.ds(base, _SCH)],
                fsems[sl]).wait()

        def spmem_step(j, sl):
            wait_flush(1 - sl)
            start_stage(j + 1, 1 - sl)
            wait_stage(sl)
            start_flush(j, sl)

        start_stage(0, 0)
        start_gather(0, 0)
        start_stage(1, 1)
        start_gather(1, 1)
        wait_stage(0)
        start_flush(0, 0)
        wait_gather(0)
        start_write(0, 0)

        @pl.loop(0, n_loop)
        def _(g):
            for t in range(2):
                stream_step(1 + 2 * g + t, (1 + t) % 2)
                spmem_step(1 + 2 * g + t, (1 + t) % 2)

        c_last = n_chunks - 1
        wait_write(0)
        wait_gather(1)
        start_write(c_last, 1)
        j_last = s_chunks - 1
        wait_stage(1)
        start_flush(j_last, 1)
        wait_write(1)
        wait_flush(0)
        wait_flush(1)

    return pl.kernel(
        body,
        out_type=jax.ShapeDtypeStruct((n_rows, dim), jnp.float32),
        mesh=mesh,
        scratch_types=[
            pltpu.VMEM((rows_per_w,), jnp.int32),
            pltpu.VMEM((2, _CH, dim), jnp.float32),
            pltpu.VMEM_SHARED((ns, 2, _SCH, dim), jnp.float32),
        ] + [pltpu.SemaphoreType.DMA] * 8,
    )


def kernel(input_ids, attention_mask, table):
    b, s = input_ids.shape
    dim = table.shape[1]
    ids = input_ids.reshape(-1).astype(jnp.int32)
    out = _build_gather(b * s, dim)(ids, table)
    return out.reshape(b, s, dim), attention_mask
